# baseline (speedup 1.0000x reference)
import jax
import jax.numpy as jnp
from jax.experimental import pallas as pl
from jax.experimental.pallas import tpu as pltpu
from jax.experimental.pallas import tpu_sc as plsc

_TAU0 = 0.5
_BB = 2
_BSC = 10
_LANES = 16
_NCH1 = 4
_CH2 = 1024

_C = (1.4426950216293335, -0.7213515043258667, 0.48091503977775574,
      -0.36030879616737366, 0.28739529848098755, -0.24826580286026,
      0.2313171774148941, -0.1443701684474945)


def _softlog2(u):
    i = jax.lax.bitcast_convert_type(u, jnp.int32)
    eb = jax.lax.shift_right_logical(i, 23)
    m = jax.lax.bitcast_convert_type(
        jnp.bitwise_or(jnp.bitwise_and(i, jnp.int32(0x007FFFFF)),
                       jnp.int32(0x3F800000)), jnp.float32)
    ef = jax.lax.bitcast_convert_type(
        jnp.bitwise_or(eb, jnp.int32(0x4B000000)), jnp.float32) - (8388608.0 + 127.0)
    big = m > 1.4142135623730951
    m = jnp.where(big, m * 0.5, m)
    ef = jnp.where(big, ef + 1.0, ef)
    y = m - 1.0
    p = jnp.full(u.shape, _C[7], dtype=jnp.float32)
    for c in _C[6::-1]:
        p = p * y + c
    return ef + y * p


def _tc_body(logits_ref, u_ref, out_ref):
    for i in range(_BB):
        l = logits_ref[i]
        u = u_ref[i]
        e2l = jnp.exp(l * (1.0 / _TAU0))
        t = jnp.log(u)
        w = e2l / (t * t)
        s = jnp.sum(w, axis=-1, keepdims=True)
        out_ref[i] = jnp.max(w * (1.0 / s), axis=0, keepdims=True)


def kernel(logits, uniform):
    B, D = logits.shape
    _, K, _ = uniform.shape
    rows = _BSC * K
    CH1 = D // _NCH1
    vmesh = plsc.VectorSubcoreMesh(core_axis_name="c", subcore_axis_name="s")

    u_rows = uniform.reshape(B * K, D)

    @pl.kernel(out_type=jax.ShapeDtypeStruct((rows, _NCH1, _LANES), jnp.float32),
               mesh=vmesh,
               scratch_types=[pltpu.VMEM((1, _LANES), jnp.float32)])
    def _sc_pass1(u_hbm, l_hbm, o_hbm, acc_ref):
        def body(u_vmem, l_vmem, o_vmem):
            acc_ref[...] = jnp.zeros((1, _LANES), jnp.float32)

            @pl.loop(0, CH1, step=_LANES)
            def _(j):
                slc = (slice(0, 1), pl.ds(j, _LANES))
                l16 = l_vmem[slc]
                e2l = jnp.exp(l16 + l16)
                t2 = _softlog2(u_vmem[slc])
                acc_ref[...] = acc_ref[...] + e2l / (t2 * t2)

            o_vmem[0] = acc_ref[...]

        pltpu.emit_pipeline(
            body,
            grid=(rows, _NCH1),
            in_specs=[pl.BlockSpec((1, CH1), lambda i, c: (i, c)),
                      pl.BlockSpec((1, CH1), lambda i, c: (i // K, c))],
            out_specs=[pl.BlockSpec((1, 1, _LANES), lambda i, c: (i, c, 0))],
            core_axis_name=("c", "s"),
            dimension_semantics=(pltpu.PARALLEL, pltpu.PARALLEL),
        )(u_hbm, l_hbm, o_hbm)

    s16 = _sc_pass1(u_rows, logits)
    s = jnp.sum(s16.reshape(rows, _NCH1 * _LANES), axis=-1)
    r16 = jnp.broadcast_to((1.0 / s).reshape(_BSC, K)[:, :, None],
                           (_BSC, K, _LANES))

    @pl.kernel(out_type=jax.ShapeDtypeStruct((_BSC, D), jnp.float32),
               mesh=vmesh)
    def _sc_pass2(u_hbm, l_hbm, r_hbm, o_hbm):
        def body(u_vmem, l_vmem, r_vmem, o_vmem):
            @pl.loop(0, _CH2, step=_LANES)
            def _(j):
                slc = (slice(0, 1), pl.ds(j, _LANES))
                l16 = l_vmem[slc]
                e2l = jnp.exp(l16 + l16)
                acc = jnp.zeros((1, _LANES), jnp.float32)
                for k in range(K):
                    t2 = _softlog2(u_vmem[0, k:k + 1, pl.ds(j, _LANES)])
                    acc = jnp.maximum(acc, (e2l / (t2 * t2)) * r_vmem[0, k:k + 1, :])
                o_vmem[slc] = acc

        pltpu.emit_pipeline(
            body,
            grid=(_BSC, D // _CH2),
            in_specs=[pl.BlockSpec((1, K, _CH2), lambda b, c: (b, 0, c)),
                      pl.BlockSpec((1, _CH2), lambda b, c: (b, c)),
                      pl.BlockSpec((1, K, _LANES), lambda b, c: (b, 0, 0))],
            out_specs=[pl.BlockSpec((1, _CH2), lambda b, c: (b, c))],
            core_axis_name=("c", "s"),
            dimension_semantics=(pltpu.PARALLEL, pltpu.PARALLEL),
        )(u_hbm, l_hbm, r_hbm, o_hbm)

    sc_out = _sc_pass2(uniform, logits, r16)

    tc_out = pl.pallas_call(
        _tc_body,
        grid=((B - _BSC) // _BB,),
        in_specs=[
            pl.BlockSpec((_BB, 1, D), lambda b: (b + _BSC // _BB, 0, 0)),
            pl.BlockSpec((_BB, K, D), lambda b: (b + _BSC // _BB, 0, 0)),
        ],
        out_specs=pl.BlockSpec((_BB, 1, D), lambda b: (b, 0, 0)),
        out_shape=jax.ShapeDtypeStruct((B - _BSC, 1, D), jnp.float32),
        compiler_params=pltpu.CompilerParams(
            dimension_semantics=("parallel",),
            vmem_limit_bytes=100 * 1024 * 1024,
        ),
    )(logits.reshape(B, 1, D), uniform)

    return jnp.concatenate([sc_out, tc_out.reshape(B - _BSC, D)], axis=0)


# speedup vs baseline: 1.7040x; 1.7040x over previous
import jax
import jax.numpy as jnp
from jax.experimental import pallas as pl
from jax.experimental.pallas import tpu as pltpu
from jax.experimental.pallas import tpu_sc as plsc

_TAU0 = 0.5
_BB = 2
_BSC = 2
_LANES = 16
_NCH1 = 4
_CH2 = 1024

_C = (1.4426950216293335, -0.7213515043258667, 0.48091503977775574,
      -0.36030879616737366, 0.28739529848098755, -0.24826580286026,
      0.2313171774148941, -0.1443701684474945)


def _softlog2(u):
    i = jax.lax.bitcast_convert_type(u, jnp.int32)
    eb = jax.lax.shift_right_logical(i, 23)
    m = jax.lax.bitcast_convert_type(
        jnp.bitwise_or(jnp.bitwise_and(i, jnp.int32(0x007FFFFF)),
                       jnp.int32(0x3F800000)), jnp.float32)
    ef = jax.lax.bitcast_convert_type(
        jnp.bitwise_or(eb, jnp.int32(0x4B000000)), jnp.float32) - (8388608.0 + 127.0)
    big = m > 1.4142135623730951
    m = jnp.where(big, m * 0.5, m)
    ef = jnp.where(big, ef + 1.0, ef)
    y = m - 1.0
    p = jnp.full(u.shape, _C[7], dtype=jnp.float32)
    for c in _C[6::-1]:
        p = p * y + c
    return ef + y * p


def _tc_body(logits_ref, u_ref, out_ref):
    for i in range(_BB):
        l = logits_ref[i]
        u = u_ref[i]
        e2l = jnp.exp(l * (1.0 / _TAU0))
        t = jnp.log(u)
        w = e2l / (t * t)
        s = jnp.sum(w, axis=-1, keepdims=True)
        out_ref[i] = jnp.max(w * (1.0 / s), axis=0, keepdims=True)


def kernel(logits, uniform):
    B, D = logits.shape
    _, K, _ = uniform.shape
    rows = _BSC * K
    CH1 = D // _NCH1
    vmesh = plsc.VectorSubcoreMesh(core_axis_name="c", subcore_axis_name="s")

    u_rows = uniform.reshape(B * K, D)

    @pl.kernel(out_type=jax.ShapeDtypeStruct((rows, _NCH1, _LANES), jnp.float32),
               mesh=vmesh,
               scratch_types=[pltpu.VMEM((1, _LANES), jnp.float32)])
    def _sc_pass1(u_hbm, l_hbm, o_hbm, acc_ref):
        def body(u_vmem, l_vmem, o_vmem):
            acc_ref[...] = jnp.zeros((1, _LANES), jnp.float32)

            @pl.loop(0, CH1, step=_LANES)
            def _(j):
                slc = (slice(0, 1), pl.ds(j, _LANES))
                l16 = l_vmem[slc]
                e2l = jnp.exp(l16 + l16)
                t2 = _softlog2(u_vmem[slc])
                acc_ref[...] = acc_ref[...] + e2l / (t2 * t2)

            o_vmem[0] = acc_ref[...]

        pltpu.emit_pipeline(
            body,
            grid=(rows, _NCH1),
            in_specs=[pl.BlockSpec((1, CH1), lambda i, c: (i, c)),
                      pl.BlockSpec((1, CH1), lambda i, c: (i // K, c))],
            out_specs=[pl.BlockSpec((1, 1, _LANES), lambda i, c: (i, c, 0))],
            core_axis_name=("c", "s"),
            dimension_semantics=(pltpu.PARALLEL, pltpu.PARALLEL),
        )(u_hbm, l_hbm, o_hbm)

    s16 = _sc_pass1(u_rows, logits)
    s = jnp.sum(s16.reshape(rows, _NCH1 * _LANES), axis=-1)
    r16 = jnp.broadcast_to((1.0 / s).reshape(_BSC, K)[:, :, None],
                           (_BSC, K, _LANES))

    @pl.kernel(out_type=jax.ShapeDtypeStruct((_BSC, D), jnp.float32),
               mesh=vmesh)
    def _sc_pass2(u_hbm, l_hbm, r_hbm, o_hbm):
        def body(u_vmem, l_vmem, r_vmem, o_vmem):
            @pl.loop(0, _CH2, step=_LANES)
            def _(j):
                slc = (slice(0, 1), pl.ds(j, _LANES))
                l16 = l_vmem[slc]
                e2l = jnp.exp(l16 + l16)
                acc = jnp.zeros((1, _LANES), jnp.float32)
                for k in range(K):
                    t2 = _softlog2(u_vmem[0, k:k + 1, pl.ds(j, _LANES)])
                    acc = jnp.maximum(acc, (e2l / (t2 * t2)) * r_vmem[0, k:k + 1, :])
                o_vmem[slc] = acc

        pltpu.emit_pipeline(
            body,
            grid=(_BSC, D // _CH2),
            in_specs=[pl.BlockSpec((1, K, _CH2), lambda b, c: (b, 0, c)),
                      pl.BlockSpec((1, _CH2), lambda b, c: (b, c)),
                      pl.BlockSpec((1, K, _LANES), lambda b, c: (b, 0, 0))],
            out_specs=[pl.BlockSpec((1, _CH2), lambda b, c: (b, c))],
            core_axis_name=("c", "s"),
            dimension_semantics=(pltpu.PARALLEL, pltpu.PARALLEL),
        )(u_hbm, l_hbm, r_hbm, o_hbm)

    sc_out = _sc_pass2(uniform, logits, r16)

    tc_out = pl.pallas_call(
        _tc_body,
        grid=((B - _BSC) // _BB,),
        in_specs=[
            pl.BlockSpec((_BB, 1, D), lambda b: (b + _BSC // _BB, 0, 0)),
            pl.BlockSpec((_BB, K, D), lambda b: (b + _BSC // _BB, 0, 0)),
        ],
        out_specs=pl.BlockSpec((_BB, 1, D), lambda b: (b, 0, 0)),
        out_shape=jax.ShapeDtypeStruct((B - _BSC, 1, D), jnp.float32),
        compiler_params=pltpu.CompilerParams(
            dimension_semantics=("parallel",),
            vmem_limit_bytes=100 * 1024 * 1024,
        ),
    )(logits.reshape(B, 1, D), uniform)

    return jnp.concatenate([sc_out, tc_out.reshape(B - _BSC, D)], axis=0)


# speedup vs baseline: 3.7543x; 2.2033x over previous
import jax
import jax.numpy as jnp
from jax.experimental import pallas as pl
from jax.experimental.pallas import tpu as pltpu

_TAU0 = 0.5
_BB = 2


def _body(logits_ref, u_ref, out_ref):
    for i in range(_BB):
        l = logits_ref[i]
        u = u_ref[i]
        e2l = jnp.exp(l * (1.0 / _TAU0))
        t = jnp.log(u)
        w = e2l / (t * t)
        s = jnp.sum(w, axis=-1, keepdims=True)
        out_ref[i] = jnp.max(w * (1.0 / s), axis=0, keepdims=True)


def kernel(logits, uniform):
    B, D = logits.shape
    _, K, _ = uniform.shape
    out = pl.pallas_call(
        _body,
        grid=(B // _BB,),
        in_specs=[
            pl.BlockSpec((_BB, 1, D), lambda b: (b, 0, 0)),
            pl.BlockSpec((_BB, K, D), lambda b: (b, 0, 0)),
        ],
        out_specs=pl.BlockSpec((_BB, 1, D), lambda b: (b, 0, 0)),
        out_shape=jax.ShapeDtypeStruct((B, 1, D), jnp.float32),
        compiler_params=pltpu.CompilerParams(
            dimension_semantics=("parallel",),
            vmem_limit_bytes=100 * 1024 * 1024,
        ),
    )(logits.reshape(B, 1, D), uniform)
    return out.reshape(B, D)
